# baseline (device time: 17020 ns/iter reference)
import jax
import jax.numpy as jnp
from jax import lax
from jax.experimental import pallas as pl
from jax.experimental.pallas import tpu as pltpu

N_Z = 4


def kernel(x):
    m, n = x.shape
    blk = n // N_Z
    qm = m // 4

    def body(x_ref, out_ref, send_z, recv_z, send_p, recv_p):
        mx = lax.axis_index("x")
        my = lax.axis_index("y")
        mz = lax.axis_index("z")
        q = 2 * mx + my
        qrow = qm * q
        peers = [(1 - mx, my), (mx, 1 - my), (1 - mx, 1 - my)]

        barrier_sem = pltpu.get_barrier_semaphore()
        for d in range(1, N_Z):
            pl.semaphore_signal(
                barrier_sem,
                inc=1,
                device_id=(mx, my, lax.rem(mz + d, N_Z)),
                device_id_type=pl.DeviceIdType.MESH,
            )
        for px, py in peers:
            pl.semaphore_signal(
                barrier_sem,
                inc=1,
                device_id=(px, py, mz),
                device_id_type=pl.DeviceIdType.MESH,
            )
        pl.semaphore_wait(barrier_sem, 6)

        for d in range(1, N_Z):
            dz = lax.rem(mz + d, N_Z)
            rdma = pltpu.make_async_remote_copy(
                src_ref=x_ref.at[pl.ds(qrow, qm), pl.ds(dz * blk, blk)],
                dst_ref=out_ref.at[pl.ds(mz * m + qrow, qm), :],
                send_sem=send_z.at[d - 1],
                recv_sem=recv_z.at[d - 1],
                device_id=(mx, my, dz),
                device_id_type=pl.DeviceIdType.MESH,
            )
            rdma.start()

        out_ref[pl.ds(mz * m, m), :] = x_ref[:, pl.ds(mz * blk, blk)]

        for k in range(N_Z):

            @pl.when(mz == k)
            def _(k=k):
                d_order = sorted(range(1, N_Z), key=lambda d: abs(k - (k - d) % N_Z))
                for d in d_order:
                    sz = (k - d) % N_Z
                    recv = pltpu.make_async_remote_copy(
                        src_ref=x_ref.at[pl.ds(0, qm), pl.ds(0, blk)],
                        dst_ref=out_ref.at[pl.ds(sz * m + qrow, qm), :],
                        send_sem=send_z.at[d - 1],
                        recv_sem=recv_z.at[d - 1],
                        device_id=(mx, my, sz),
                        device_id_type=pl.DeviceIdType.MESH,
                    )
                    recv.wait_recv()
                    for r, (px, py) in enumerate(peers):
                        fwd = pltpu.make_async_remote_copy(
                            src_ref=out_ref.at[pl.ds(sz * m + qrow, qm), :],
                            dst_ref=out_ref.at[pl.ds(sz * m + qrow, qm), :],
                            send_sem=send_p.at[r, d - 1],
                            recv_sem=recv_p.at[r, d - 1],
                            device_id=(px, py, k),
                            device_id_type=pl.DeviceIdType.MESH,
                        )
                        fwd.start()
                for d in d_order:
                    sz = (k - d) % N_Z
                    for r, (px, py) in enumerate(peers):
                        pq = 2 * px + py
                        recv = pltpu.make_async_remote_copy(
                            src_ref=x_ref.at[pl.ds(0, qm), pl.ds(0, blk)],
                            dst_ref=out_ref.at[pl.ds(sz * m + qm * pq, qm), :],
                            send_sem=send_p.at[r, d - 1],
                            recv_sem=recv_p.at[r, d - 1],
                            device_id=(px, py, k),
                            device_id_type=pl.DeviceIdType.MESH,
                        )
                        recv.wait_recv()

        for d in range(1, N_Z):
            drain = pltpu.make_async_remote_copy(
                src_ref=x_ref.at[pl.ds(0, qm), pl.ds(0, blk)],
                dst_ref=out_ref.at[pl.ds(0, qm), :],
                send_sem=send_z.at[d - 1],
                recv_sem=recv_z.at[d - 1],
                device_id=(mx, my, mz),
                device_id_type=pl.DeviceIdType.MESH,
            )
            drain.wait_send()
            for r in range(3):
                drain = pltpu.make_async_remote_copy(
                    src_ref=x_ref.at[pl.ds(0, qm), pl.ds(0, blk)],
                    dst_ref=out_ref.at[pl.ds(0, qm), :],
                    send_sem=send_p.at[r, d - 1],
                    recv_sem=recv_p.at[r, d - 1],
                    device_id=(mx, my, mz),
                    device_id_type=pl.DeviceIdType.MESH,
                )
                drain.wait_send()

    out_shape = jax.ShapeDtypeStruct((N_Z * m, blk), jnp.float32)
    return pl.pallas_call(
        body,
        out_shape=out_shape,
        in_specs=[pl.BlockSpec(memory_space=pltpu.VMEM)],
        out_specs=pl.BlockSpec(memory_space=pltpu.VMEM),
        scratch_shapes=[
            pltpu.SemaphoreType.DMA((N_Z - 1,)),
            pltpu.SemaphoreType.DMA((N_Z - 1,)),
            pltpu.SemaphoreType.DMA((3, N_Z - 1)),
            pltpu.SemaphoreType.DMA((3, N_Z - 1)),
        ],
        compiler_params=pltpu.CompilerParams(collective_id=0),
    )(x)


# device time: 16193 ns/iter; 1.0511x vs baseline; 1.0511x over previous
import jax
import jax.numpy as jnp
from jax import lax
from jax.experimental import pallas as pl
from jax.experimental.pallas import tpu as pltpu

N_Z = 4


def kernel(x):
    m, n = x.shape
    blk = n // N_Z

    def body(x_ref, out_ref, send_sems, recv_sems):
        my_x = lax.axis_index("x")
        my_y = lax.axis_index("y")
        my_z = lax.axis_index("z")

        barrier_sem = pltpu.get_barrier_semaphore()
        for d in range(1, N_Z):
            peer = lax.rem(my_z + d, N_Z)
            pl.semaphore_signal(
                barrier_sem,
                inc=1,
                device_id=(my_x, my_y, peer),
                device_id_type=pl.DeviceIdType.MESH,
            )
        pl.semaphore_wait(barrier_sem, N_Z - 1)

        sends = []
        for d in range(1, N_Z):
            peer = lax.rem(my_z + d, N_Z)
            rdma = pltpu.make_async_remote_copy(
                src_ref=x_ref.at[:, pl.ds(peer * blk, blk)],
                dst_ref=out_ref.at[pl.ds(my_z * m, m), :],
                send_sem=send_sems.at[d - 1],
                recv_sem=recv_sems.at[d - 1],
                device_id=(my_x, my_y, peer),
                device_id_type=pl.DeviceIdType.MESH,
            )
            rdma.start()
            sends.append(rdma)

        out_ref[pl.ds(my_z * m, m), :] = x_ref[:, pl.ds(my_z * blk, blk)]

        for d in range(1, N_Z):
            src_z = lax.rem(my_z - d + N_Z, N_Z)
            recv = pltpu.make_async_remote_copy(
                src_ref=x_ref.at[:, pl.ds(0, blk)],
                dst_ref=out_ref.at[pl.ds(src_z * m, m), :],
                send_sem=send_sems.at[d - 1],
                recv_sem=recv_sems.at[d - 1],
                device_id=(my_x, my_y, src_z),
                device_id_type=pl.DeviceIdType.MESH,
            )
            recv.wait_recv()

        for rdma in sends:
            rdma.wait_send()

    out_shape = jax.ShapeDtypeStruct((N_Z * m, blk), jnp.float32)
    return pl.pallas_call(
        body,
        out_shape=out_shape,
        in_specs=[pl.BlockSpec(memory_space=pltpu.VMEM)],
        out_specs=pl.BlockSpec(memory_space=pltpu.VMEM),
        scratch_shapes=[
            pltpu.SemaphoreType.DMA((N_Z - 1,)),
            pltpu.SemaphoreType.DMA((N_Z - 1,)),
        ],
        compiler_params=pltpu.CompilerParams(collective_id=0),
    )(x)
